# Initial kernel scaffold; baseline (speedup 1.0000x reference)
import jax
import jax.numpy as jnp
from jax.experimental import pallas as pl


def kernel(logits):
    raise NotImplementedError("write your pallas kernel here")


# speedup vs baseline: 19.2874x; 19.2874x over previous
import functools

import jax
import jax.numpy as jnp
from jax import lax
from jax.experimental import pallas as pl
from jax.experimental.pallas import tpu as pltpu
from jax.experimental.pallas import tpu_sc as plsc

B = 128
C = 32768
TOPK = 10
CAND = 16
GAP_T = 0.01
NEG = -3.0e38

NUM_WORKERS = 32
ROWS_PER_W = B // NUM_WORKERS
NCHUNK = C // 16
UNROLL = 8


def _sc_topk_body(logits_hbm, out_hbm, buf0, buf1, stage, sem0, sem1):
    nc = 2
    cidx = lax.axis_index("c")
    sidx = lax.axis_index("s")
    wid = sidx * nc + cidx
    base = wid * ROWS_PER_W
    bufs = (buf0, buf1)
    sems = (sem0, sem1)

    copies = [None, None]
    copies[0] = pltpu.async_copy(logits_hbm.at[base], buf0, sem0)
    for r in range(ROWS_PER_W):
        if r + 1 < ROWS_PER_W:
            copies[(r + 1) % 2] = pltpu.async_copy(
                logits_hbm.at[base + r + 1], bufs[(r + 1) % 2], sems[(r + 1) % 2])
        copies[r % 2].wait()
        ref = bufs[r % 2]

        def p1(i, m):
            for u in range(UNROLL):
                m = jnp.maximum(m, ref[pl.ds((i * UNROLL + u) * 16, 16)])
            return m

        lane_max = lax.fori_loop(0, NCHUNK // UNROLL, p1,
                                 jnp.full((16,), NEG, jnp.float32))
        tvec = lane_max
        for k in (8, 4, 2, 1):
            idx = lax.iota(jnp.int32, 16) ^ k
            tvec = jnp.minimum(
                tvec, tvec.at[idx].get(mode="promise_in_bounds"))

        def p2(i, cand):
            for u in range(UNROLL):
                x = ref[pl.ds((i * UNROLL + u) * 16, 16)]
                msk = x >= tvec

                def do_merge(c):
                    sx = jnp.sort(jnp.where(msk, x, NEG))
                    merged = jnp.maximum(c, jnp.flip(sx))
                    return jnp.sort(merged)

                cand = lax.cond(jnp.any(msk), do_merge, lambda c: c, cand)
            return cand

        cand = lax.fori_loop(0, NCHUNK // UNROLL, p2,
                             jnp.full((16,), NEG, jnp.float32))
        stage[...] = jnp.flip(cand)
        pltpu.sync_copy(stage, out_hbm.at[base + r])


def _sc_topk(logits):
    mesh = plsc.VectorSubcoreMesh(core_axis_name="c", subcore_axis_name="s")
    return pl.kernel(
        _sc_topk_body,
        mesh=mesh,
        compiler_params=pltpu.CompilerParams(needs_layout_passes=False),
        out_type=jax.ShapeDtypeStruct((B, CAND), jnp.float32),
        scratch_types=[
            pltpu.VMEM((C,), jnp.float32),
            pltpu.VMEM((C,), jnp.float32),
            pltpu.VMEM((CAND,), jnp.float32),
            pltpu.SemaphoreType.DMA,
            pltpu.SemaphoreType.DMA,
        ],
    )(logits)


def _tc_stats_kernel(x_ref, m_ref, z_ref, s1_ref, a_ref):
    x = x_ref[...]
    m = jnp.max(x, axis=1, keepdims=True)
    xm = x - m
    e = jnp.exp(xm)
    z = jnp.sum(e, axis=1, keepdims=True)
    s1 = jnp.sum(xm * e, axis=1, keepdims=True)
    cols = lax.broadcasted_iota(jnp.int32, x.shape, 1)
    am = jnp.min(jnp.where(x == m, cols, C), axis=1, keepdims=True)
    m_ref[...] = m.reshape(1, 8, 1)
    z_ref[...] = z.reshape(1, 8, 1)
    s1_ref[...] = s1.reshape(1, 8, 1)
    a_ref[...] = am.reshape(1, 8, 1)


def _tc_stats(logits):
    n = B // 8
    o3 = jax.ShapeDtypeStruct((n, 8, 1), jnp.float32)
    oi = jax.ShapeDtypeStruct((n, 8, 1), jnp.int32)
    spec3 = pl.BlockSpec((1, 8, 1), lambda i: (i, 0, 0))
    m, z, s1, am = pl.pallas_call(
        _tc_stats_kernel,
        grid=(n,),
        in_specs=[pl.BlockSpec((8, C), lambda i: (i, 0))],
        out_specs=[spec3, spec3, spec3, spec3],
        out_shape=[o3, o3, o3, oi],
    )(logits)
    return (m.reshape(B, 1), z.reshape(B, 1), s1.reshape(B, 1),
            am.reshape(B, 1))


def _epilogue_kernel(m_ref, z_ref, zr_ref, s1_ref, a_ref, ar_ref, t_ref,
                     out_ref):
    m = m_ref[...]
    z = z_ref[...]
    z_row = zr_ref[...]
    s1 = s1_ref[...]
    am = a_ref[...]
    am_row = ar_ref[...]
    t = t_ref[...]

    eq = (am == am_row).astype(jnp.float32)
    z_col = jnp.mean(eq, axis=1, keepdims=True)
    z_rw = jnp.mean(eq, axis=0, keepdims=True)
    z_bar = jnp.mean(eq)
    bias_col = z_bar - z_col
    bias_row = z_bar - z_rw
    mask_col = bias_col >= 0.0
    mask_row = bias_row >= 0.0
    nm = jnp.sum(mask_row.astype(jnp.float32))

    def qrank(x_col, x_row):
        lo = jnp.sum(jnp.where(mask_row & (x_row < x_col), 1.0, 0.0),
                     axis=1, keepdims=True)
        hi = jnp.sum(jnp.where(mask_row & (x_row <= x_col), 1.0, 0.0),
                     axis=1, keepdims=True)
        q = ((lo + 1.0 + hi) * 0.5) / jnp.maximum(nm, 1.0)
        return jnp.where(mask_col, q, 0.0)

    conf_col = 1.0 / z
    conf_row = 1.0 / z_row
    q_z = qrank(bias_col, bias_row)
    q_k = qrank(conf_col, conf_row)

    ent_std = jnp.log(z) - s1 / z

    lane = lax.broadcasted_iota(jnp.int32, t.shape, 1)
    valid = lane < TOPK
    ek = jnp.where(valid, jnp.exp(t - t[:, 0:1]), 0.0)
    zk = jnp.sum(ek, axis=1, keepdims=True)
    p = ek / zk
    ent_topk = -jnp.sum(jnp.where(valid, p * jnp.log(p + 1e-8), 0.0),
                        axis=1, keepdims=True)

    et = jnp.where(valid, jnp.exp(t - m), 0.0)
    zt = z - jnp.sum(et, axis=1, keepdims=True)
    s1t = s1 - jnp.sum(jnp.where(valid, (t - m) * et, 0.0),
                       axis=1, keepdims=True)
    ent_tail = jnp.log(zt) - s1t / zt

    gap = (1.0 - jnp.exp(t[:, 1:2] - m)) / z
    high_conf = gap > GAP_T

    weights = jnp.where(mask_col, q_z * q_k, -0.5)
    fe = jnp.where(high_conf, ent_std,
                   jnp.where(mask_col, ent_topk, ent_tail))
    out_ref[...] = jnp.mean(weights * fe).reshape(1, 1)


def _epilogue(m, z, s1, am, cand):
    return pl.pallas_call(
        _epilogue_kernel,
        out_shape=jax.ShapeDtypeStruct((1, 1), jnp.float32),
    )(m, z, z.reshape(1, B), s1, am, am.reshape(1, B), cand)


def kernel(logits):
    cand = _sc_topk(logits)
    m, z, s1, am = _tc_stats(logits)
    loss = _epilogue(m, z, s1, am, cand)
    return loss[0, 0]


# speedup vs baseline: 31.0532x; 1.6100x over previous
import functools

import jax
import jax.numpy as jnp
from jax import lax
from jax.experimental import pallas as pl
from jax.experimental.pallas import tpu as pltpu
from jax.experimental.pallas import tpu_sc as plsc

B = 128
C = 32768
TOPK = 10
CAND = 16
GAP_T = 0.01
NEG = -3.0e38

CHUNK = 128
NCH = C // CHUNK
NGRP = NCH // 16

NUM_WORKERS = 32
ROWS_PER_W = B // NUM_WORKERS


def _sc_topk_body(tbl_hbm, cm_hbm, out_hbm, cmbuf, idxbuf, gbuf, stage, sem):
    nc = 2
    cidx = lax.axis_index("c")
    sidx = lax.axis_index("s")
    wid = sidx * nc + cidx
    base = wid * ROWS_PER_W

    pltpu.sync_copy(cm_hbm.at[pl.ds(base, ROWS_PER_W)], cmbuf)

    iota = lax.iota(jnp.int32, 16)
    izero = jnp.zeros((16,), jnp.int32)
    tvecs = []
    for r in range(ROWS_PER_W):
        keys = jnp.full((16,), NEG, jnp.float32)
        vals = jnp.zeros((16,), jnp.int32)
        for g in range(NGRP):
            k = cmbuf[r, pl.ds(g * 16, 16)]
            v = iota + g * 16
            bk, bv = plsc.sort_key_val(k, v, descending=True)
            take = keys >= bk
            mk = jnp.where(take, keys, bk)
            mv = jnp.where(take, vals, bv)
            keys, vals = plsc.sort_key_val(mk, mv)
        tvecs.append(keys.at[izero].get(mode="promise_in_bounds"))
        idxbuf[pl.ds(r * 16, 16)] = vals + (base + r) * NCH

    pltpu.async_copy(tbl_hbm.at[idxbuf], gbuf, sem).wait()

    for r in range(ROWS_PER_W):
        tvec = tvecs[r]

        def scan(i, cand, r=r, tvec=tvec):
            for u in range(CHUNK // 16):
                x = gbuf[r * 16 + i, pl.ds(u * 16, 16)]
                msk = x >= tvec

                def do_merge(c):
                    sx = jnp.sort(jnp.where(msk, x, NEG))
                    merged = jnp.maximum(c, lax.rev(sx, (0,)))
                    return jnp.sort(merged)

                cand = lax.cond(jnp.any(msk), do_merge, lambda c: c, cand)
            return cand

        cand = lax.fori_loop(0, 16, scan,
                             jnp.full((16,), NEG, jnp.float32))
        stage[r, pl.ds(0, 16)] = lax.rev(cand, (0,))

    pltpu.sync_copy(stage, out_hbm.at[pl.ds(base, ROWS_PER_W)])


def _sc_topk(tbl, cm):
    mesh = plsc.VectorSubcoreMesh(core_axis_name="c", subcore_axis_name="s")
    return pl.kernel(
        _sc_topk_body,
        mesh=mesh,
        compiler_params=pltpu.CompilerParams(needs_layout_passes=False),
        out_type=jax.ShapeDtypeStruct((B, CAND), jnp.float32),
        scratch_types=[
            pltpu.VMEM((ROWS_PER_W, NCH), jnp.float32),
            pltpu.VMEM((ROWS_PER_W * 16,), jnp.int32),
            pltpu.VMEM((ROWS_PER_W * 16, CHUNK), jnp.float32),
            pltpu.VMEM((ROWS_PER_W, CAND), jnp.float32),
            pltpu.SemaphoreType.DMA,
        ],
    )(tbl, cm)


def _tc_stats_kernel(x_ref, m_ref, z_ref, s1_ref, a_ref, cm_ref):
    x = x_ref[...]
    cm = jnp.max(x.reshape(8, NCH, CHUNK), axis=2)
    m = jnp.max(cm, axis=1, keepdims=True)
    xm = x - m
    e = jnp.exp(xm)
    z = jnp.sum(e, axis=1, keepdims=True)
    s1 = jnp.sum(xm * e, axis=1, keepdims=True)
    cols = lax.broadcasted_iota(jnp.int32, x.shape, 1)
    am = jnp.min(jnp.where(x == m, cols, C), axis=1, keepdims=True)
    m_ref[...] = m.reshape(1, 8, 1)
    z_ref[...] = z.reshape(1, 8, 1)
    s1_ref[...] = s1.reshape(1, 8, 1)
    a_ref[...] = am.reshape(1, 8, 1)
    cm_ref[...] = cm.reshape(1, 8, NCH)


def _tc_stats(logits):
    n = B // 8
    o3 = jax.ShapeDtypeStruct((n, 8, 1), jnp.float32)
    oi = jax.ShapeDtypeStruct((n, 8, 1), jnp.int32)
    oc = jax.ShapeDtypeStruct((n, 8, NCH), jnp.float32)
    spec3 = pl.BlockSpec((1, 8, 1), lambda i: (i, 0, 0))
    specc = pl.BlockSpec((1, 8, NCH), lambda i: (i, 0, 0))
    m, z, s1, am, cm = pl.pallas_call(
        _tc_stats_kernel,
        grid=(n,),
        in_specs=[pl.BlockSpec((8, C), lambda i: (i, 0))],
        out_specs=[spec3, spec3, spec3, spec3, specc],
        out_shape=[o3, o3, o3, oi, oc],
    )(logits)
    return (m.reshape(B, 1), z.reshape(B, 1), s1.reshape(B, 1),
            am.reshape(B, 1), cm.reshape(B, NCH))


def _epilogue_kernel(m_ref, z_ref, zr_ref, s1_ref, a_ref, ar_ref, t_ref,
                     out_ref):
    m = m_ref[...]
    z = z_ref[...]
    z_row = zr_ref[...]
    s1 = s1_ref[...]
    am = a_ref[...]
    am_row = ar_ref[...]
    t = t_ref[...]

    eq = (am == am_row).astype(jnp.float32)
    z_col = jnp.mean(eq, axis=1, keepdims=True)
    z_rw = jnp.mean(eq, axis=0, keepdims=True)
    z_bar = jnp.mean(eq)
    bias_col = z_bar - z_col
    bias_row = z_bar - z_rw
    mask_col = bias_col >= 0.0
    mask_row = bias_row >= 0.0
    nm = jnp.sum(mask_row.astype(jnp.float32))

    def qrank(x_col, x_row):
        lo = jnp.sum(jnp.where(mask_row & (x_row < x_col), 1.0, 0.0),
                     axis=1, keepdims=True)
        hi = jnp.sum(jnp.where(mask_row & (x_row <= x_col), 1.0, 0.0),
                     axis=1, keepdims=True)
        q = ((lo + 1.0 + hi) * 0.5) / jnp.maximum(nm, 1.0)
        return jnp.where(mask_col, q, 0.0)

    conf_col = 1.0 / z
    conf_row = 1.0 / z_row
    q_z = qrank(bias_col, bias_row)
    q_k = qrank(conf_col, conf_row)

    ent_std = jnp.log(z) - s1 / z

    lane = lax.broadcasted_iota(jnp.int32, t.shape, 1)
    valid = lane < TOPK
    ek = jnp.where(valid, jnp.exp(t - t[:, 0:1]), 0.0)
    zk = jnp.sum(ek, axis=1, keepdims=True)
    p = ek / zk
    ent_topk = -jnp.sum(jnp.where(valid, p * jnp.log(p + 1e-8), 0.0),
                        axis=1, keepdims=True)

    et = jnp.where(valid, jnp.exp(t - m), 0.0)
    zt = z - jnp.sum(et, axis=1, keepdims=True)
    s1t = s1 - jnp.sum(jnp.where(valid, (t - m) * et, 0.0),
                       axis=1, keepdims=True)
    ent_tail = jnp.log(zt) - s1t / zt

    gap = (1.0 - jnp.exp(t[:, 1:2] - m)) / z
    high_conf = gap > GAP_T

    weights = jnp.where(mask_col, q_z * q_k, -0.5)
    fe = jnp.where(high_conf, ent_std,
                   jnp.where(mask_col, ent_topk, ent_tail))
    out_ref[...] = jnp.mean(weights * fe).reshape(1, 1)


def _epilogue(m, z, s1, am, cand):
    return pl.pallas_call(
        _epilogue_kernel,
        out_shape=jax.ShapeDtypeStruct((1, 1), jnp.float32),
    )(m, z, z.reshape(1, B), s1, am, am.reshape(1, B), cand)


def kernel(logits):
    m, z, s1, am, cm = _tc_stats(logits)
    cand = _sc_topk(logits.reshape(B * NCH, CHUNK), cm)
    loss = _epilogue(m, z, s1, am, cand)
    return loss[0, 0]


# speedup vs baseline: 39.2644x; 1.2644x over previous
import functools

import jax
import jax.numpy as jnp
from jax import lax
from jax.experimental import pallas as pl
from jax.experimental.pallas import tpu as pltpu
from jax.experimental.pallas import tpu_sc as plsc

B = 128
C = 32768
TOPK = 10
CAND = 16
GAP_T = 0.01
NEG = -3.0e38

CHUNK = 128
NCH = C // CHUNK
NGRP = NCH // 16

NUM_WORKERS = 32
ROWS_PER_W = B // NUM_WORKERS


def _sc_topk_body(logits_hbm, cm_hbm, out_hbm, cmbuf, buf0, buf1, stage,
                  sem0, sem1):
    nc = 2
    cidx = lax.axis_index("c")
    sidx = lax.axis_index("s")
    wid = sidx * nc + cidx
    base = wid * ROWS_PER_W

    pltpu.sync_copy(cm_hbm.at[pl.ds(base, ROWS_PER_W)], cmbuf)

    iota = lax.iota(jnp.int32, 16)
    izero = jnp.zeros((16,), jnp.int32)
    tvecs = []
    offs = []
    for r in range(ROWS_PER_W):
        keys = jnp.full((16,), NEG, jnp.float32)
        vals = jnp.zeros((16,), jnp.int32)
        for g in range(NGRP):
            k = cmbuf[r, pl.ds(g * 16, 16)]
            v = iota + g * 16
            bk, bv = plsc.sort_key_val(k, v, descending=True)
            take = keys >= bk
            mk = jnp.where(take, keys, bk)
            mv = jnp.where(take, vals, bv)
            keys, vals = plsc.sort_key_val(mk, mv)
        tvecs.append(keys.at[izero].get(mode="promise_in_bounds"))
        offs.append(vals * CHUNK)

    bufs = (buf0, buf1)
    sems = (sem0, sem1)
    copies = [pltpu.async_copy(logits_hbm.at[base], buf0, sem0), None]
    for r in range(ROWS_PER_W):
        if r + 1 < ROWS_PER_W:
            copies[(r + 1) % 2] = pltpu.async_copy(
                logits_hbm.at[base + r + 1], bufs[(r + 1) % 2],
                sems[(r + 1) % 2])
        copies[r % 2].wait()
        ref = bufs[r % 2]
        tvec = tvecs[r]
        off = offs[r]

        def scan(o, cand, ref=ref, tvec=tvec, off=off):
            for u in range(4):
                x = plsc.load_gather(ref, [off + (o * 4 + u)])
                msk = x >= tvec

                def do_merge(c):
                    sx = jnp.sort(jnp.where(msk, x, NEG))
                    merged = jnp.maximum(c, lax.rev(sx, (0,)))
                    return jnp.sort(merged)

                cand = lax.cond(jnp.any(msk), do_merge, lambda c: c, cand)
            return cand

        cand = lax.fori_loop(0, CHUNK // 4, scan,
                             jnp.full((16,), NEG, jnp.float32))
        stage[r, pl.ds(0, 16)] = lax.rev(cand, (0,))

    pltpu.sync_copy(stage, out_hbm.at[pl.ds(base, ROWS_PER_W)])


def _sc_topk(logits, cm):
    mesh = plsc.VectorSubcoreMesh(core_axis_name="c", subcore_axis_name="s")
    return pl.kernel(
        _sc_topk_body,
        mesh=mesh,
        compiler_params=pltpu.CompilerParams(needs_layout_passes=False),
        out_type=jax.ShapeDtypeStruct((B, CAND), jnp.float32),
        scratch_types=[
            pltpu.VMEM((ROWS_PER_W, NCH), jnp.float32),
            pltpu.VMEM((C,), jnp.float32),
            pltpu.VMEM((C,), jnp.float32),
            pltpu.VMEM((ROWS_PER_W, CAND), jnp.float32),
            pltpu.SemaphoreType.DMA,
            pltpu.SemaphoreType.DMA,
        ],
    )(logits, cm)


def _tc_stats_kernel(x_ref, m_ref, z_ref, s1_ref, a_ref, cm_ref):
    x = x_ref[...]
    cm = jnp.max(x.reshape(8, NCH, CHUNK), axis=2)
    m = jnp.max(cm, axis=1, keepdims=True)
    xm = x - m
    e = jnp.exp(xm)
    z = jnp.sum(e, axis=1, keepdims=True)
    s1 = jnp.sum(xm * e, axis=1, keepdims=True)
    cols = lax.broadcasted_iota(jnp.int32, x.shape, 1)
    am = jnp.min(jnp.where(x == m, cols, C), axis=1, keepdims=True)
    m_ref[...] = m
    z_ref[...] = z
    s1_ref[...] = s1
    a_ref[...] = am
    cm_ref[...] = cm


def _tc_stats(logits):
    n = B // 8
    o2 = jax.ShapeDtypeStruct((B, 1), jnp.float32)
    oi = jax.ShapeDtypeStruct((B, 1), jnp.int32)
    oc = jax.ShapeDtypeStruct((B, NCH), jnp.float32)
    spec2 = pl.BlockSpec((8, 1), lambda i: (i, 0))
    specc = pl.BlockSpec((8, NCH), lambda i: (i, 0))
    return pl.pallas_call(
        _tc_stats_kernel,
        grid=(n,),
        in_specs=[pl.BlockSpec((8, C), lambda i: (i, 0))],
        out_specs=[spec2, spec2, spec2, spec2, specc],
        out_shape=[o2, o2, o2, oi, oc],
    )(logits)


def _epilogue_kernel(m_ref, z_ref, zr_ref, s1_ref, a_ref, ar_ref, t_ref,
                     out_ref):
    m = m_ref[...]
    z = z_ref[...]
    z_row = zr_ref[...]
    s1 = s1_ref[...]
    am = a_ref[...]
    am_row = ar_ref[...]
    t = t_ref[...]

    eq = (am == am_row).astype(jnp.float32)
    z_col = jnp.mean(eq, axis=1, keepdims=True)
    z_rw = jnp.mean(eq, axis=0, keepdims=True)
    z_bar = jnp.mean(eq)
    bias_col = z_bar - z_col
    bias_row = z_bar - z_rw
    mask_col = bias_col >= 0.0
    mask_row = bias_row >= 0.0
    nm = jnp.sum(mask_row.astype(jnp.float32))

    def qrank(x_col, x_row):
        lo = jnp.sum(jnp.where(mask_row & (x_row < x_col), 1.0, 0.0),
                     axis=1, keepdims=True)
        hi = jnp.sum(jnp.where(mask_row & (x_row <= x_col), 1.0, 0.0),
                     axis=1, keepdims=True)
        q = ((lo + 1.0 + hi) * 0.5) / jnp.maximum(nm, 1.0)
        return jnp.where(mask_col, q, 0.0)

    conf_col = 1.0 / z
    conf_row = 1.0 / z_row
    q_z = qrank(bias_col, bias_row)
    q_k = qrank(conf_col, conf_row)

    ent_std = jnp.log(z) - s1 / z

    lane = lax.broadcasted_iota(jnp.int32, t.shape, 1)
    valid = lane < TOPK
    ek = jnp.where(valid, jnp.exp(t - t[:, 0:1]), 0.0)
    zk = jnp.sum(ek, axis=1, keepdims=True)
    p = ek / zk
    ent_topk = -jnp.sum(jnp.where(valid, p * jnp.log(p + 1e-8), 0.0),
                        axis=1, keepdims=True)

    et = jnp.where(valid, jnp.exp(t - m), 0.0)
    zt = z - jnp.sum(et, axis=1, keepdims=True)
    s1t = s1 - jnp.sum(jnp.where(valid, (t - m) * et, 0.0),
                       axis=1, keepdims=True)
    ent_tail = jnp.log(zt) - s1t / zt

    gap = (1.0 - jnp.exp(t[:, 1:2] - m)) / z
    high_conf = gap > GAP_T

    weights = jnp.where(mask_col, q_z * q_k, -0.5)
    fe = jnp.where(high_conf, ent_std,
                   jnp.where(mask_col, ent_topk, ent_tail))
    out_ref[...] = jnp.mean(weights * fe).reshape(1, 1)


def _epilogue(m, z, s1, am, cand):
    return pl.pallas_call(
        _epilogue_kernel,
        out_shape=jax.ShapeDtypeStruct((1, 1), jnp.float32),
    )(m, z, z.reshape(1, B), s1, am, am.reshape(1, B), cand)


def kernel(logits):
    m, z, s1, am, cm = _tc_stats(logits)
    cand = _sc_topk(logits, cm)
    loss = _epilogue(m, z, s1, am, cand)
    return loss[0, 0]


# speedup vs baseline: 39.3363x; 1.0018x over previous
import functools

import jax
import jax.numpy as jnp
from jax import lax
from jax.experimental import pallas as pl
from jax.experimental.pallas import tpu as pltpu
from jax.experimental.pallas import tpu_sc as plsc

B = 128
C = 32768
TOPK = 10
CAND = 16
GAP_T = 0.01
NEG = -3.0e38

CHUNK = 128
NCH = C // CHUNK
NGRP = NCH // 16

NUM_WORKERS = 32
ROWS_PER_W = B // NUM_WORKERS


def _sc_topk_body(logits_hbm, cm_hbm, out_hbm, cmbuf, buf0, buf1,
                  stage, sem0, sem1):
    nc = 2
    cidx = lax.axis_index("c")
    sidx = lax.axis_index("s")
    wid = sidx * nc + cidx
    base = wid * ROWS_PER_W

    pltpu.sync_copy(cm_hbm.at[pl.ds(base, ROWS_PER_W)], cmbuf)

    iota = lax.iota(jnp.int32, 16)
    izero = jnp.zeros((16,), jnp.int32)
    tvecs = []
    offs = []
    for r in range(ROWS_PER_W):
        keys = jnp.full((16,), NEG, jnp.float32)
        vals = jnp.zeros((16,), jnp.int32)
        for g in range(NGRP):
            k = cmbuf[r, pl.ds(g * 16, 16)]
            v = (iota + g * 16) * CHUNK
            bk, bv = plsc.sort_key_val(k, v, descending=True)
            take = keys >= bk
            mk = jnp.where(take, keys, bk)
            mv = jnp.where(take, vals, bv)
            keys, vals = plsc.sort_key_val(mk, mv)
        tvecs.append(keys.at[izero].get(mode="promise_in_bounds"))
        offs.append(vals)

    bufs = (buf0, buf1)
    sems = (sem0, sem1)
    copies = [pltpu.async_copy(logits_hbm.at[base], buf0, sem0), None]
    for r in range(ROWS_PER_W):
        if r + 1 < ROWS_PER_W:
            copies[(r + 1) % 2] = pltpu.async_copy(
                logits_hbm.at[base + r + 1], bufs[(r + 1) % 2],
                sems[(r + 1) % 2])
        copies[r % 2].wait()
        ref = bufs[r % 2]
        tvec = tvecs[r]
        off = offs[r]

        def scan(o, cand, ref=ref, tvec=tvec, off=off):
            for u in range(4):
                x = plsc.load_gather(ref, [off + (o * 4 + u)])
                msk = x >= tvec

                def do_merge(c):
                    sx = jnp.sort(jnp.where(msk, x, NEG))
                    merged = jnp.maximum(c, lax.rev(sx, (0,)))
                    return jnp.sort(merged)

                cand = lax.cond(jnp.any(msk), do_merge, lambda c: c, cand)
            return cand

        cand = lax.fori_loop(0, CHUNK // 4, scan,
                             jnp.full((16,), NEG, jnp.float32))
        stage[r, pl.ds(0, 16)] = lax.rev(cand, (0,))

    pltpu.sync_copy(stage, out_hbm.at[pl.ds(base, ROWS_PER_W)])


def _sc_topk(logits, cm):
    mesh = plsc.VectorSubcoreMesh(core_axis_name="c", subcore_axis_name="s")
    return pl.kernel(
        _sc_topk_body,
        mesh=mesh,
        compiler_params=pltpu.CompilerParams(needs_layout_passes=False),
        out_type=jax.ShapeDtypeStruct((B, CAND), jnp.float32),
        scratch_types=[
            pltpu.VMEM((ROWS_PER_W, NCH), jnp.float32),
            pltpu.VMEM((C,), jnp.float32),
            pltpu.VMEM((C,), jnp.float32),
            pltpu.VMEM((ROWS_PER_W, CAND), jnp.float32),
            pltpu.SemaphoreType.DMA,
            pltpu.SemaphoreType.DMA,
        ],
    )(logits, cm)


def _tc_stats_kernel(x_ref, m_ref, z_ref, s1_ref, a_ref, cm_ref):
    x = x_ref[...]
    cm = jnp.max(x.reshape(8, NCH, CHUNK), axis=2)
    m = jnp.max(cm, axis=1, keepdims=True)
    xm = x - m
    e = jnp.exp(xm)
    z = jnp.sum(e, axis=1, keepdims=True)
    s1 = jnp.sum(xm * e, axis=1, keepdims=True)
    cols = lax.broadcasted_iota(jnp.int32, x.shape, 1)
    am = jnp.min(jnp.where(x == m, cols, C), axis=1, keepdims=True)
    m_ref[...] = m
    z_ref[...] = z
    s1_ref[...] = s1
    a_ref[...] = am
    cm_ref[...] = cm


def _tc_stats(logits):
    n = B // 8
    o2 = jax.ShapeDtypeStruct((B, 1), jnp.float32)
    oi = jax.ShapeDtypeStruct((B, 1), jnp.int32)
    oc = jax.ShapeDtypeStruct((B, NCH), jnp.float32)
    spec2 = pl.BlockSpec((8, 1), lambda i: (i, 0))
    specc = pl.BlockSpec((8, NCH), lambda i: (i, 0))
    return pl.pallas_call(
        _tc_stats_kernel,
        grid=(n,),
        in_specs=[pl.BlockSpec((8, C), lambda i: (i, 0))],
        out_specs=[spec2, spec2, spec2, spec2, specc],
        out_shape=[o2, o2, o2, oi, oc],
    )(logits)


def _epilogue_kernel(m_ref, z_ref, zr_ref, s1_ref, a_ref, ar_ref, t_ref,
                     out_ref):
    m = m_ref[...]
    z = z_ref[...]
    z_row = zr_ref[...]
    s1 = s1_ref[...]
    am = a_ref[...]
    am_row = ar_ref[...]
    t = t_ref[...]

    eq = (am == am_row).astype(jnp.float32)
    z_col = jnp.mean(eq, axis=1, keepdims=True)
    z_rw = jnp.mean(eq, axis=0, keepdims=True)
    z_bar = jnp.mean(eq)
    bias_col = z_bar - z_col
    bias_row = z_bar - z_rw
    mask_col = bias_col >= 0.0
    mask_row = bias_row >= 0.0
    nm = jnp.sum(mask_row.astype(jnp.float32))

    def qrank(x_col, x_row):
        lo = jnp.sum(jnp.where(mask_row & (x_row < x_col), 1.0, 0.0),
                     axis=1, keepdims=True)
        hi = jnp.sum(jnp.where(mask_row & (x_row <= x_col), 1.0, 0.0),
                     axis=1, keepdims=True)
        q = ((lo + 1.0 + hi) * 0.5) / jnp.maximum(nm, 1.0)
        return jnp.where(mask_col, q, 0.0)

    conf_col = 1.0 / z
    conf_row = 1.0 / z_row
    q_z = qrank(bias_col, bias_row)
    q_k = qrank(conf_col, conf_row)

    ent_std = jnp.log(z) - s1 / z

    lane = lax.broadcasted_iota(jnp.int32, t.shape, 1)
    valid = lane < TOPK
    ek = jnp.where(valid, jnp.exp(t - t[:, 0:1]), 0.0)
    zk = jnp.sum(ek, axis=1, keepdims=True)
    p = ek / zk
    ent_topk = -jnp.sum(jnp.where(valid, p * jnp.log(p + 1e-8), 0.0),
                        axis=1, keepdims=True)

    et = jnp.where(valid, jnp.exp(t - m), 0.0)
    zt = z - jnp.sum(et, axis=1, keepdims=True)
    s1t = s1 - jnp.sum(jnp.where(valid, (t - m) * et, 0.0),
                       axis=1, keepdims=True)
    ent_tail = jnp.log(zt) - s1t / zt

    gap = (1.0 - jnp.exp(t[:, 1:2] - m)) / z
    high_conf = gap > GAP_T

    weights = jnp.where(mask_col, q_z * q_k, -0.5)
    fe = jnp.where(high_conf, ent_std,
                   jnp.where(mask_col, ent_topk, ent_tail))
    out_ref[...] = jnp.mean(weights * fe).reshape(1, 1)


def _epilogue(m, z, s1, am, cand):
    return pl.pallas_call(
        _epilogue_kernel,
        out_shape=jax.ShapeDtypeStruct((1, 1), jnp.float32),
    )(m, z, z.reshape(1, B), s1, am, am.reshape(1, B), cand)


def kernel(logits):
    m, z, s1, am, cm = _tc_stats(logits)
    cand = _sc_topk(logits, cm)
    loss = _epilogue(m, z, s1, am, cand)
    return loss[0, 0]


# speedup vs baseline: 44.2544x; 1.1250x over previous
import functools

import jax
import jax.numpy as jnp
from jax import lax
from jax.experimental import pallas as pl
from jax.experimental.pallas import tpu as pltpu
from jax.experimental.pallas import tpu_sc as plsc

B = 128
C = 32768
TOPK = 10
CAND = 16
GAP_T = 0.01
NEG = -3.0e38

CHUNK = 128
NCH = C // CHUNK
NGRP = NCH // 16

NUM_WORKERS = 32
ROWS_PER_W = B // NUM_WORKERS


def _sc_topk_body(logits_hbm, cm_hbm, out_hbm, cmbuf, buf0, buf1,
                  stage, sem0, sem1):
    nc = 2
    cidx = lax.axis_index("c")
    sidx = lax.axis_index("s")
    wid = sidx * nc + cidx
    base = wid * ROWS_PER_W

    pltpu.sync_copy(cm_hbm.at[pl.ds(base, ROWS_PER_W)], cmbuf)

    iota = lax.iota(jnp.int32, 16)
    izero = jnp.zeros((16,), jnp.int32)
    tvecs = []
    offs = []
    for r in range(ROWS_PER_W):
        keys = jnp.full((16,), NEG, jnp.float32)
        vals = jnp.zeros((16,), jnp.int32)
        for g in range(NGRP):
            k = cmbuf[r, pl.ds(g * 16, 16)]
            v = iota + g * 16
            bk, bv = plsc.sort_key_val(k, v, descending=True)
            take = keys >= bk
            mk = jnp.where(take, keys, bk)
            mv = jnp.where(take, vals, bv)
            keys, vals = plsc.sort_key_val(mk, mv)
        tvecs.append(keys.at[izero].get(mode="promise_in_bounds"))
        offs.append(vals)

    bufs = (buf0, buf1)
    sems = (sem0, sem1)
    copies = [pltpu.async_copy(logits_hbm.at[base], buf0, sem0), None]
    for r in range(ROWS_PER_W):
        if r + 1 < ROWS_PER_W:
            copies[(r + 1) % 2] = pltpu.async_copy(
                logits_hbm.at[base + r + 1], bufs[(r + 1) % 2],
                sems[(r + 1) % 2])
        copies[r % 2].wait()
        ref = bufs[r % 2]
        tvec = tvecs[r]
        off = offs[r]

        def scan(o, cand, ref=ref, tvec=tvec, off=off):
            for u in range(4):
                x = plsc.load_gather(ref, [off + (o * 4 + u) * NCH])
                msk = x >= tvec

                def do_merge(c):
                    sx = jnp.sort(jnp.where(msk, x, NEG))
                    merged = jnp.maximum(c, lax.rev(sx, (0,)))
                    return jnp.sort(merged)

                cand = lax.cond(jnp.any(msk), do_merge, lambda c: c, cand)
            return cand

        cand = lax.fori_loop(0, CHUNK // 4, scan,
                             jnp.full((16,), NEG, jnp.float32))
        stage[r, pl.ds(0, 16)] = lax.rev(cand, (0,))

    pltpu.sync_copy(stage, out_hbm.at[pl.ds(base, ROWS_PER_W)])


def _sc_topk(logits, cm):
    mesh = plsc.VectorSubcoreMesh(core_axis_name="c", subcore_axis_name="s")
    return pl.kernel(
        _sc_topk_body,
        mesh=mesh,
        compiler_params=pltpu.CompilerParams(needs_layout_passes=False),
        out_type=jax.ShapeDtypeStruct((B, CAND), jnp.float32),
        scratch_types=[
            pltpu.VMEM((ROWS_PER_W, NCH), jnp.float32),
            pltpu.VMEM((C,), jnp.float32),
            pltpu.VMEM((C,), jnp.float32),
            pltpu.VMEM((ROWS_PER_W, CAND), jnp.float32),
            pltpu.SemaphoreType.DMA,
            pltpu.SemaphoreType.DMA,
        ],
    )(logits, cm)


def _tc_stats_kernel(x_ref, m_ref, z_ref, s1_ref, a_ref, cm_ref):
    x = x_ref[...]
    cm = x[:, 0:NCH]
    for s in range(1, CHUNK):
        cm = jnp.maximum(cm, x[:, s * NCH:(s + 1) * NCH])
    m = jnp.max(cm, axis=1, keepdims=True)
    zp = jnp.zeros((8, NCH), jnp.float32)
    s1p = jnp.zeros((8, NCH), jnp.float32)
    for s in range(CHUNK):
        sl = x[:, s * NCH:(s + 1) * NCH] - m
        e = jnp.exp(sl)
        zp = zp + e
        s1p = s1p + sl * e
    z = jnp.sum(zp, axis=1, keepdims=True)
    s1 = jnp.sum(s1p, axis=1, keepdims=True)
    cols = lax.broadcasted_iota(jnp.int32, x.shape, 1)
    am = jnp.min(jnp.where(x == m, cols, C), axis=1, keepdims=True)
    m_ref[...] = m
    z_ref[...] = z
    s1_ref[...] = s1
    a_ref[...] = am
    cm_ref[...] = cm


def _tc_stats(logits):
    n = B // 8
    o2 = jax.ShapeDtypeStruct((B, 1), jnp.float32)
    oi = jax.ShapeDtypeStruct((B, 1), jnp.int32)
    oc = jax.ShapeDtypeStruct((B, NCH), jnp.float32)
    spec2 = pl.BlockSpec((8, 1), lambda i: (i, 0))
    specc = pl.BlockSpec((8, NCH), lambda i: (i, 0))
    return pl.pallas_call(
        _tc_stats_kernel,
        grid=(n,),
        in_specs=[pl.BlockSpec((8, C), lambda i: (i, 0))],
        out_specs=[spec2, spec2, spec2, spec2, specc],
        out_shape=[o2, o2, o2, oi, oc],
    )(logits)


def _epilogue_kernel(m_ref, z_ref, zr_ref, s1_ref, a_ref, ar_ref, t_ref,
                     out_ref):
    m = m_ref[...]
    z = z_ref[...]
    z_row = zr_ref[...]
    s1 = s1_ref[...]
    am = a_ref[...]
    am_row = ar_ref[...]
    t = t_ref[...]

    eq = (am == am_row).astype(jnp.float32)
    z_col = jnp.mean(eq, axis=1, keepdims=True)
    z_rw = jnp.mean(eq, axis=0, keepdims=True)
    z_bar = jnp.mean(eq)
    bias_col = z_bar - z_col
    bias_row = z_bar - z_rw
    mask_col = bias_col >= 0.0
    mask_row = bias_row >= 0.0
    nm = jnp.sum(mask_row.astype(jnp.float32))

    def qrank(x_col, x_row):
        lo = jnp.sum(jnp.where(mask_row & (x_row < x_col), 1.0, 0.0),
                     axis=1, keepdims=True)
        hi = jnp.sum(jnp.where(mask_row & (x_row <= x_col), 1.0, 0.0),
                     axis=1, keepdims=True)
        q = ((lo + 1.0 + hi) * 0.5) / jnp.maximum(nm, 1.0)
        return jnp.where(mask_col, q, 0.0)

    conf_col = 1.0 / z
    conf_row = 1.0 / z_row
    q_z = qrank(bias_col, bias_row)
    q_k = qrank(conf_col, conf_row)

    ent_std = jnp.log(z) - s1 / z

    lane = lax.broadcasted_iota(jnp.int32, t.shape, 1)
    valid = lane < TOPK
    ek = jnp.where(valid, jnp.exp(t - t[:, 0:1]), 0.0)
    zk = jnp.sum(ek, axis=1, keepdims=True)
    p = ek / zk
    ent_topk = -jnp.sum(jnp.where(valid, p * jnp.log(p + 1e-8), 0.0),
                        axis=1, keepdims=True)

    et = jnp.where(valid, jnp.exp(t - m), 0.0)
    zt = z - jnp.sum(et, axis=1, keepdims=True)
    s1t = s1 - jnp.sum(jnp.where(valid, (t - m) * et, 0.0),
                       axis=1, keepdims=True)
    ent_tail = jnp.log(zt) - s1t / zt

    gap = (1.0 - jnp.exp(t[:, 1:2] - m)) / z
    high_conf = gap > GAP_T

    weights = jnp.where(mask_col, q_z * q_k, -0.5)
    fe = jnp.where(high_conf, ent_std,
                   jnp.where(mask_col, ent_topk, ent_tail))
    out_ref[...] = jnp.mean(weights * fe).reshape(1, 1)


def _epilogue(m, z, s1, am, cand):
    return pl.pallas_call(
        _epilogue_kernel,
        out_shape=jax.ShapeDtypeStruct((1, 1), jnp.float32),
    )(m, z, z.reshape(1, B), s1, am, am.reshape(1, B), cand)


def kernel(logits):
    m, z, s1, am, cm = _tc_stats(logits)
    cand = _sc_topk(logits, cm)
    loss = _epilogue(m, z, s1, am, cand)
    return loss[0, 0]


# speedup vs baseline: 45.7184x; 1.0331x over previous
import functools

import jax
import jax.numpy as jnp
from jax import lax
from jax.experimental import pallas as pl
from jax.experimental.pallas import tpu as pltpu
from jax.experimental.pallas import tpu_sc as plsc

B = 128
C = 32768
TOPK = 10
CAND = 16
GAP_T = 0.01
NEG = -3.0e38

CHUNK = 128
NCH = C // CHUNK
NGRP = NCH // 16

NUM_WORKERS = 32
ROWS_PER_W = B // NUM_WORKERS


def _sc_topk_body(logits_hbm, cm_hbm, out_hbm, cmbuf, buf0, buf1,
                  stage, sem0, sem1):
    nc = 2
    cidx = lax.axis_index("c")
    sidx = lax.axis_index("s")
    wid = sidx * nc + cidx
    base = wid * ROWS_PER_W

    pltpu.sync_copy(cm_hbm.at[pl.ds(base, ROWS_PER_W)], cmbuf)

    iota = lax.iota(jnp.int32, 16)
    izero = jnp.zeros((16,), jnp.int32)
    tvecs = []
    offs = []
    for r in range(ROWS_PER_W):
        keys = jnp.full((16,), NEG, jnp.float32)
        vals = jnp.zeros((16,), jnp.int32)
        for g in range(NGRP):
            k = cmbuf[r, pl.ds(g * 16, 16)]
            v = iota + g * 16
            bk, bv = plsc.sort_key_val(k, v, descending=True)
            take = keys >= bk
            mk = jnp.where(take, keys, bk)
            mv = jnp.where(take, vals, bv)
            keys, vals = plsc.sort_key_val(mk, mv)
        tvecs.append(keys.at[izero].get(mode="promise_in_bounds"))
        offs.append(vals)

    bufs = (buf0, buf1)
    sems = (sem0, sem1)
    copies = [pltpu.async_copy(logits_hbm.at[base], buf0, sem0), None]
    for r in range(ROWS_PER_W):
        if r + 1 < ROWS_PER_W:
            copies[(r + 1) % 2] = pltpu.async_copy(
                logits_hbm.at[base + r + 1], bufs[(r + 1) % 2],
                sems[(r + 1) % 2])
        copies[r % 2].wait()
        ref = bufs[r % 2]
        tvec = tvecs[r]
        off = offs[r]

        def scan(o, cand, ref=ref, tvec=tvec, off=off):
            for u in range(4):
                x = plsc.load_gather(ref, [off + (o * 4 + u) * NCH])
                msk = x >= tvec

                def do_merge(c):
                    sx = jnp.sort(jnp.where(msk, x, NEG))
                    merged = jnp.maximum(c, lax.rev(sx, (0,)))
                    return jnp.sort(merged)

                cand = lax.cond(jnp.any(msk), do_merge, lambda c: c, cand)
            return cand

        cand = lax.fori_loop(0, CHUNK // 4, scan,
                             jnp.full((16,), NEG, jnp.float32))
        stage[r, pl.ds(0, 16)] = lax.rev(cand, (0,))

    pltpu.sync_copy(stage, out_hbm.at[pl.ds(base, ROWS_PER_W)])


def _sc_topk(logits, cm):
    mesh = plsc.VectorSubcoreMesh(core_axis_name="c", subcore_axis_name="s")
    return pl.kernel(
        _sc_topk_body,
        mesh=mesh,
        compiler_params=pltpu.CompilerParams(needs_layout_passes=False),
        out_type=jax.ShapeDtypeStruct((B, CAND), jnp.float32),
        scratch_types=[
            pltpu.VMEM((ROWS_PER_W, NCH), jnp.float32),
            pltpu.VMEM((C,), jnp.float32),
            pltpu.VMEM((C,), jnp.float32),
            pltpu.VMEM((ROWS_PER_W, CAND), jnp.float32),
            pltpu.SemaphoreType.DMA,
            pltpu.SemaphoreType.DMA,
        ],
    )(logits, cm)


def _tc_stats_kernel(x_ref, m_ref, z_ref, s1_ref, a_ref, cm_ref):
    x = x_ref[...]
    cm = x[:, 0:NCH]
    sa = jnp.zeros((8, NCH), jnp.int32)
    for s in range(1, CHUNK):
        sl = x[:, s * NCH:(s + 1) * NCH]
        sa = jnp.where(sl > cm, s, sa)
        cm = jnp.maximum(cm, sl)
    m = jnp.max(cm, axis=1, keepdims=True)
    zp = jnp.zeros((8, NCH), jnp.float32)
    s1p = jnp.zeros((8, NCH), jnp.float32)
    for s in range(CHUNK):
        sl = x[:, s * NCH:(s + 1) * NCH] - m
        e = jnp.exp(sl)
        zp = zp + e
        s1p = s1p + sl * e
    z = jnp.sum(zp, axis=1, keepdims=True)
    s1 = jnp.sum(s1p, axis=1, keepdims=True)
    cols = lax.broadcasted_iota(jnp.int32, (8, NCH), 1)
    am = jnp.min(jnp.where(cm == m, sa * NCH + cols, C),
                 axis=1, keepdims=True)
    m_ref[...] = m
    z_ref[...] = z
    s1_ref[...] = s1
    a_ref[...] = am
    cm_ref[...] = cm


def _tc_stats(logits):
    n = B // 8
    o2 = jax.ShapeDtypeStruct((B, 1), jnp.float32)
    oi = jax.ShapeDtypeStruct((B, 1), jnp.int32)
    oc = jax.ShapeDtypeStruct((B, NCH), jnp.float32)
    spec2 = pl.BlockSpec((8, 1), lambda i: (i, 0))
    specc = pl.BlockSpec((8, NCH), lambda i: (i, 0))
    return pl.pallas_call(
        _tc_stats_kernel,
        grid=(n,),
        in_specs=[pl.BlockSpec((8, C), lambda i: (i, 0))],
        out_specs=[spec2, spec2, spec2, spec2, specc],
        out_shape=[o2, o2, o2, oi, oc],
    )(logits)


def _epilogue_kernel(m_ref, z_ref, zr_ref, s1_ref, a_ref, ar_ref, t_ref,
                     out_ref):
    m = m_ref[...]
    z = z_ref[...]
    z_row = zr_ref[...]
    s1 = s1_ref[...]
    am = a_ref[...]
    am_row = ar_ref[...]
    t = t_ref[...]

    eq = (am == am_row).astype(jnp.float32)
    z_col = jnp.mean(eq, axis=1, keepdims=True)
    z_rw = jnp.mean(eq, axis=0, keepdims=True)
    z_bar = jnp.mean(eq)
    bias_col = z_bar - z_col
    bias_row = z_bar - z_rw
    mask_col = bias_col >= 0.0
    mask_row = bias_row >= 0.0
    nm = jnp.sum(mask_row.astype(jnp.float32))

    def qrank(x_col, x_row):
        lo = jnp.sum(jnp.where(mask_row & (x_row < x_col), 1.0, 0.0),
                     axis=1, keepdims=True)
        hi = jnp.sum(jnp.where(mask_row & (x_row <= x_col), 1.0, 0.0),
                     axis=1, keepdims=True)
        q = ((lo + 1.0 + hi) * 0.5) / jnp.maximum(nm, 1.0)
        return jnp.where(mask_col, q, 0.0)

    conf_col = 1.0 / z
    conf_row = 1.0 / z_row
    q_z = qrank(bias_col, bias_row)
    q_k = qrank(conf_col, conf_row)

    ent_std = jnp.log(z) - s1 / z

    lane = lax.broadcasted_iota(jnp.int32, t.shape, 1)
    valid = lane < TOPK
    ek = jnp.where(valid, jnp.exp(t - t[:, 0:1]), 0.0)
    zk = jnp.sum(ek, axis=1, keepdims=True)
    p = ek / zk
    ent_topk = -jnp.sum(jnp.where(valid, p * jnp.log(p + 1e-8), 0.0),
                        axis=1, keepdims=True)

    et = jnp.where(valid, jnp.exp(t - m), 0.0)
    zt = z - jnp.sum(et, axis=1, keepdims=True)
    s1t = s1 - jnp.sum(jnp.where(valid, (t - m) * et, 0.0),
                       axis=1, keepdims=True)
    ent_tail = jnp.log(zt) - s1t / zt

    gap = (1.0 - jnp.exp(t[:, 1:2] - m)) / z
    high_conf = gap > GAP_T

    weights = jnp.where(mask_col, q_z * q_k, -0.5)
    fe = jnp.where(high_conf, ent_std,
                   jnp.where(mask_col, ent_topk, ent_tail))
    out_ref[...] = jnp.mean(weights * fe).reshape(1, 1)


def _epilogue(m, z, s1, am, cand):
    return pl.pallas_call(
        _epilogue_kernel,
        out_shape=jax.ShapeDtypeStruct((1, 1), jnp.float32),
    )(m, z, z.reshape(1, B), s1, am, am.reshape(1, B), cand)


def kernel(logits):
    m, z, s1, am, cm = _tc_stats(logits)
    cand = _sc_topk(logits, cm)
    loss = _epilogue(m, z, s1, am, cand)
    return loss[0, 0]
